# baseline (speedup 1.0000x reference)
import jax
import jax.numpy as jnp
from jax import lax
from jax.experimental import pallas as pl
from jax.experimental.pallas import tpu as pltpu
from jax.experimental.pallas import tpu_sc as plsc

B = 4096
C = 28
K = 10
CK = C * K
H = 64
G4 = 4 * H
OUT = 768
BLK = 512
NB = B // BLK

NC = 2
NS = 16
LANES = 16
NW = NC * NS
RW = B // NW
EPW = RW * C
NCHUNK = EPW // LANES
HROWS = 128
NTAB = HROWS * LANES

_mesh = plsc.VectorSubcoreMesh(core_axis_name="c", subcore_axis_name="s")
_params = pltpu.CompilerParams(needs_layout_passes=False)


def _hist_body(vs_hbm, lev_hbm, parts_hbm, vs_v, lev_v, hist_v):
    cid = lax.axis_index("c")
    sid = lax.axis_index("s")
    wid = sid * NC + cid
    pltpu.sync_copy(vs_hbm.at[pl.ds(wid * EPW, EPW)], vs_v)
    pltpu.sync_copy(lev_hbm.at[pl.ds(wid * RW, RW)], lev_v)
    iota = lax.broadcasted_iota(jnp.int32, (LANES,), 0)

    def zero_body(j, carry):
        hist_v[j, :] = jnp.zeros((LANES,), jnp.float32)
        return carry

    lax.fori_loop(0, HROWS, zero_body, 0)

    ones = jnp.ones((LANES,), jnp.float32)

    def body(i, carry):
        e = i * LANES + iota
        r = e // C
        c = e - r * C
        valf = vs_v[pl.ds(i * LANES, LANES)]
        vali = valf.astype(jnp.int32)
        lev = plsc.load_gather(lev_v, [r])
        b = lev * CK + c * K + vali
        plsc.addupdate_scatter(hist_v, [b >> 4, b & 15], ones)
        return carry

    lax.fori_loop(0, NCHUNK, body, 0)

    pltpu.sync_copy(hist_v, parts_hbm.at[pl.ds(wid * HROWS, HROWS)])


SROWS = HROWS // NS


def _weight_body(parts_hbm, vs_hbm, lev_hbm, out_hbm,
                 pslice_v, tabslice_v, tab_v, npm_v, vs_v, lev_v, out_v,
                 shared_tab, sem):
    cid = lax.axis_index("c")
    sid = lax.axis_index("s")
    wid = sid * NC + cid
    cps = [pltpu.async_copy(
               parts_hbm.at[pl.ds(w * HROWS + sid * SROWS, SROWS)],
               pslice_v.at[w], sem)
           for w in range(NW)]
    for cp in cps:
        cp.wait()
    pltpu.sync_copy(vs_hbm.at[pl.ds(wid * EPW, EPW)], vs_v)
    pltpu.sync_copy(lev_hbm.at[pl.ds(wid * RW, RW)], lev_v)
    iota = lax.broadcasted_iota(jnp.int32, (LANES,), 0)

    for jj in range(SROWS):
        acc = pslice_v[0, jj]
        for w in range(1, NW):
            acc = acc + pslice_v[w, jj]
        tabslice_v[pl.ds(jj * LANES, LANES)] = acc
    pltpu.sync_copy(tabslice_v,
                    shared_tab.at[pl.ds(sid * SROWS * LANES, SROWS * LANES)])
    plsc.subcore_barrier()
    pltpu.sync_copy(shared_tab, tab_v)

    npm = jnp.ones((LANES,), jnp.float32)
    for l in range(5):
        bi = l * CK + iota
        row = plsc.load_gather(tab_v, [bi])
        s = jnp.sum(jnp.where(iota < K, row, 0.0))
        s = jnp.maximum(s, 1.0)
        npm = jnp.where(iota == l, s, npm)
    npm_v[...] = npm

    def body(i, carry):
        e = i * LANES + iota
        r = e // C
        c = e - r * C
        valf = vs_v[pl.ds(i * LANES, LANES)]
        vali = valf.astype(jnp.int32)
        lev = plsc.load_gather(lev_v, [r])
        b = lev * CK + c * K + vali
        t = plsc.load_gather(tab_v, [b])
        d = plsc.load_gather(npm_v, [lev])
        out_v[pl.ds(i * LANES, LANES)] = valf * (t / d)
        return carry

    lax.fori_loop(0, NCHUNK, body, 0)
    pltpu.sync_copy(out_v, out_hbm.at[pl.ds(wid * EPW, EPW)])


_hist = pl.kernel(
    _hist_body,
    out_type=jax.ShapeDtypeStruct((NW * HROWS, LANES), jnp.float32),
    mesh=_mesh,
    compiler_params=_params,
    scratch_types=[
        pltpu.VMEM((EPW,), jnp.float32),
        pltpu.VMEM((RW,), jnp.int32),
        pltpu.VMEM((HROWS, LANES), jnp.float32),
    ],
)

_weight = pl.kernel(
    _weight_body,
    out_type=jax.ShapeDtypeStruct((B * C,), jnp.float32),
    mesh=_mesh,
    compiler_params=_params,
    scratch_types=[
        pltpu.VMEM((NW, SROWS, LANES), jnp.float32),
        pltpu.VMEM((SROWS * LANES,), jnp.float32),
        pltpu.VMEM((NTAB,), jnp.float32),
        pltpu.VMEM((LANES,), jnp.float32),
        pltpu.VMEM((EPW,), jnp.float32),
        pltpu.VMEM((RW,), jnp.int32),
        pltpu.VMEM((EPW,), jnp.float32),
        pltpu.VMEM_SHARED((NTAB,), jnp.float32),
        pltpu.SemaphoreType.DMA,
    ],
)


def _copy_kernel(src_ref, dst_ref):
    dst_ref[...] = src_ref[...]


def _lstm_kernel(x_ref, wih_ref, bih_ref, bhh_ref, wfc_ref, bfc_ref, out_ref):
    x = x_ref[...]
    gates = lax.dot_general(x, wih_ref[...], (((1,), (1,)), ((), ())),
                            preferred_element_type=jnp.float32)
    gates = gates + bih_ref[...] + bhh_ref[...]
    i_g = jax.nn.sigmoid(gates[:, 0:H])
    g_g = jnp.tanh(gates[:, 2 * H:3 * H])
    o_g = jax.nn.sigmoid(gates[:, 3 * H:4 * H])
    h = o_g * jnp.tanh(i_g * g_g)
    out_ref[...] = lax.dot_general(h, wfc_ref[...], (((1,), (1,)), ((), ())),
                                   preferred_element_type=jnp.float32) + bfc_ref[...]


def kernel(VS, CC, Level, Depart, W_ih, W_hh, b_ih, b_hh, W_fc, b_fc):
    vs_flat = VS.reshape(B * C)
    lev_i = Level.astype(jnp.int32)
    parts = _hist(vs_flat, lev_i)
    w_flat = _weight(parts, vs_flat, lev_i)
    x = w_flat.reshape(B, C)

    vs_feat = pl.pallas_call(
        _lstm_kernel,
        grid=(NB,),
        in_specs=[pl.BlockSpec((BLK, C), lambda i: (i, 0)),
                  pl.BlockSpec((G4, C), lambda i: (0, 0)),
                  pl.BlockSpec((1, G4), lambda i: (0, 0)),
                  pl.BlockSpec((1, G4), lambda i: (0, 0)),
                  pl.BlockSpec((OUT, H), lambda i: (0, 0)),
                  pl.BlockSpec((1, OUT), lambda i: (0, 0))],
        out_specs=pl.BlockSpec((BLK, OUT), lambda i: (i, 0)),
        out_shape=jax.ShapeDtypeStruct((B, OUT), jnp.float32),
    )(x, W_ih, b_ih.reshape(1, G4), b_hh.reshape(1, G4),
      W_fc, b_fc.reshape(1, OUT))

    cc_feat = pl.pallas_call(
        _copy_kernel,
        grid=(NB,),
        in_specs=[pl.BlockSpec((BLK, OUT), lambda i: (i, 0))],
        out_specs=pl.BlockSpec((BLK, OUT), lambda i: (i, 0)),
        out_shape=jax.ShapeDtypeStruct((B, OUT), jnp.float32),
    )(jnp.squeeze(CC, axis=1))

    return (cc_feat, vs_feat, Level, Depart)


# speedup vs baseline: 1.3673x; 1.3673x over previous
import jax
import jax.numpy as jnp
from jax import lax
from jax.experimental import pallas as pl
from jax.experimental.pallas import tpu as pltpu
from jax.experimental.pallas import tpu_sc as plsc

B = 4096
C = 28
K = 10
CK = C * K
H = 64
G4 = 4 * H
OUT = 768
BLK = 512
NB = B // BLK

NC = 2
NS = 16
LANES = 16
NW = NC * NS
RW = B // NW
EPW = RW * C
NCHUNK = EPW // LANES
HROWS = 128
NTAB = HROWS * LANES

_mesh = plsc.VectorSubcoreMesh(core_axis_name="c", subcore_axis_name="s")
_params = pltpu.CompilerParams(needs_layout_passes=False)


RPT = B // NS
EPT = RPT * C
NCHT = EPT // LANES
PSL = NTAB // NS


def _fused_body(vs_hbm, lev_hbm, out_hbm,
                vs_v, lev_v, hist_v, pslice_v, tabslice_v, tab_v, npm_v,
                shared_parts, shared_tab, sem):
    cid = lax.axis_index("c")
    sid = lax.axis_index("s")
    wid = sid * NC + cid
    pltpu.sync_copy(vs_hbm.at[pl.ds(sid * EPT, EPT)], vs_v)
    pltpu.sync_copy(lev_hbm.at[pl.ds(sid * RPT, RPT)], lev_v)
    iota = lax.broadcasted_iota(jnp.int32, (LANES,), 0)

    def zero_body(j, carry):
        hist_v[pl.ds(j * LANES, LANES)] = jnp.zeros((LANES,), jnp.float32)
        return carry

    lax.fori_loop(0, NTAB // LANES, zero_body, 0)

    ones = jnp.ones((LANES,), jnp.float32)

    def hbody(i, carry):
        e = i * LANES + iota
        r = e // C
        c = e - r * C
        valf = vs_v[pl.ds(i * LANES, LANES)]
        vali = valf.astype(jnp.int32)
        lev = plsc.load_gather(lev_v, [r])
        b = lev * CK + c * K + vali
        plsc.addupdate_scatter(hist_v, [b], ones)
        return carry

    lax.fori_loop(0, NCHT, hbody, 0)

    pltpu.sync_copy(hist_v, shared_parts.at[pl.ds(sid * NTAB, NTAB)])
    plsc.subcore_barrier()
    cps = [pltpu.async_copy(
               shared_parts.at[pl.ds(t * NTAB + sid * PSL, PSL)],
               pslice_v.at[t], sem)
           for t in range(NS)]
    for cp in cps:
        cp.wait()
    for jj in range(PSL // LANES):
        acc = pslice_v[0, pl.ds(jj * LANES, LANES)]
        for t in range(1, NS):
            acc = acc + pslice_v[t, pl.ds(jj * LANES, LANES)]
        tabslice_v[pl.ds(jj * LANES, LANES)] = acc
    pltpu.sync_copy(tabslice_v, shared_tab.at[pl.ds(sid * PSL, PSL)])
    plsc.subcore_barrier()
    pltpu.sync_copy(shared_tab, tab_v)

    npm = jnp.ones((LANES,), jnp.float32)
    for l in range(5):
        row = plsc.load_gather(tab_v, [l * CK + iota])
        s = jnp.sum(jnp.where(iota < K, row, 0.0))
        s = jnp.maximum(s, 1.0)
        npm = jnp.where(iota == l, s, npm)
    npm_v[...] = npm

    boff = cid * EPW

    def wbody(i, carry):
        o = boff + i * LANES
        e = o + iota
        r = e // C
        c = e - r * C
        valf = vs_v[pl.ds(o, LANES)]
        vali = valf.astype(jnp.int32)
        lev = plsc.load_gather(lev_v, [r])
        b = lev * CK + c * K + vali
        t = plsc.load_gather(tab_v, [b])
        d = plsc.load_gather(npm_v, [lev])
        vs_v[pl.ds(o, LANES)] = valf * (t / d)
        return carry

    lax.fori_loop(0, NCHUNK, wbody, 0)
    pltpu.sync_copy(vs_v.at[pl.ds(boff, EPW)],
                    out_hbm.at[pl.ds(wid * EPW, EPW)])


_fused = pl.kernel(
    _fused_body,
    out_type=jax.ShapeDtypeStruct((B * C,), jnp.float32),
    mesh=_mesh,
    compiler_params=_params,
    scratch_types=[
        pltpu.VMEM((EPT,), jnp.float32),
        pltpu.VMEM((RPT,), jnp.int32),
        pltpu.VMEM((NTAB,), jnp.float32),
        pltpu.VMEM((NS, PSL), jnp.float32),
        pltpu.VMEM((PSL,), jnp.float32),
        pltpu.VMEM((NTAB,), jnp.float32),
        pltpu.VMEM((LANES,), jnp.float32),
        pltpu.VMEM_SHARED((NS * NTAB,), jnp.float32),
        pltpu.VMEM_SHARED((NTAB,), jnp.float32),
        pltpu.SemaphoreType.DMA,
    ],
)


def _lstm_kernel(x_ref, wih_ref, bih_ref, bhh_ref, wfc_ref, bfc_ref, out_ref):
    x = x_ref[...]
    gates = lax.dot_general(x, wih_ref[...], (((1,), (1,)), ((), ())),
                            preferred_element_type=jnp.float32)
    gates = gates + bih_ref[...] + bhh_ref[...]
    i_g = jax.nn.sigmoid(gates[:, 0:H])
    g_g = jnp.tanh(gates[:, 2 * H:3 * H])
    o_g = jax.nn.sigmoid(gates[:, 3 * H:4 * H])
    h = o_g * jnp.tanh(i_g * g_g)
    out_ref[...] = lax.dot_general(h, wfc_ref[...], (((1,), (1,)), ((), ())),
                                   preferred_element_type=jnp.float32) + bfc_ref[...]


def kernel(VS, CC, Level, Depart, W_ih, W_hh, b_ih, b_hh, W_fc, b_fc):
    vs_flat = VS.reshape(B * C)
    lev_i = Level.astype(jnp.int32)
    w_flat = _fused(vs_flat, lev_i)
    x = w_flat.reshape(B, C)

    vs_feat = pl.pallas_call(
        _lstm_kernel,
        grid=(NB,),
        in_specs=[pl.BlockSpec((BLK, C), lambda i: (i, 0)),
                  pl.BlockSpec((G4, C), lambda i: (0, 0)),
                  pl.BlockSpec((1, G4), lambda i: (0, 0)),
                  pl.BlockSpec((1, G4), lambda i: (0, 0)),
                  pl.BlockSpec((OUT, H), lambda i: (0, 0)),
                  pl.BlockSpec((1, OUT), lambda i: (0, 0))],
        out_specs=pl.BlockSpec((BLK, OUT), lambda i: (i, 0)),
        out_shape=jax.ShapeDtypeStruct((B, OUT), jnp.float32),
    )(x, W_ih, b_ih.reshape(1, G4), b_hh.reshape(1, G4),
      W_fc, b_fc.reshape(1, OUT))

    return (jnp.squeeze(CC, axis=1), vs_feat, Level, Depart)


# speedup vs baseline: 1.3704x; 1.0023x over previous
import jax
import jax.numpy as jnp
from jax import lax
from jax.experimental import pallas as pl
from jax.experimental.pallas import tpu as pltpu
from jax.experimental.pallas import tpu_sc as plsc

B = 4096
C = 28
K = 10
CK = C * K
H = 64
G4 = 4 * H
OUT = 768
BLK = 512
NB = B // BLK

NC = 2
NS = 16
LANES = 16
NW = NC * NS
RW = B // NW
EPW = RW * C
NCHUNK = EPW // LANES
HROWS = 128
NTAB = HROWS * LANES

_mesh = plsc.VectorSubcoreMesh(core_axis_name="c", subcore_axis_name="s")
_params = pltpu.CompilerParams(needs_layout_passes=False)


RPT = B // NS
EPT = RPT * C
NCHT = EPT // LANES
PSL = NTAB // NS


def _fused_body(vs_hbm, lev_hbm, out_hbm,
                vs_v, lev_v, hist_v, pslice_v, tabslice_v, tab_v, npm_v,
                shared_parts, shared_tab, sem):
    cid = lax.axis_index("c")
    sid = lax.axis_index("s")
    wid = sid * NC + cid
    cp_vs = pltpu.async_copy(vs_hbm.at[pl.ds(sid * EPT, EPT)], vs_v, sem)
    cp_lev = pltpu.async_copy(lev_hbm.at[pl.ds(sid * RPT, RPT)], lev_v, sem)
    iota = lax.broadcasted_iota(jnp.int32, (LANES,), 0)

    def zero_body(j, carry):
        hist_v[pl.ds(j * LANES, LANES)] = jnp.zeros((LANES,), jnp.float32)
        return carry

    lax.fori_loop(0, NTAB // LANES, zero_body, 0)
    cp_vs.wait()
    cp_lev.wait()

    ones = jnp.ones((LANES,), jnp.float32)

    def hbody(i, carry):
        for u in range(2):
            o = (2 * i + u) * LANES
            e = o + iota
            r = e // C
            c = e - r * C
            valf = vs_v[pl.ds(o, LANES)]
            vali = valf.astype(jnp.int32)
            lev = plsc.load_gather(lev_v, [r])
            b = lev * CK + c * K + vali
            plsc.addupdate_scatter(hist_v, [b], ones)
        return carry

    lax.fori_loop(0, NCHT // 2, hbody, 0)

    pltpu.sync_copy(hist_v, shared_parts.at[pl.ds(sid * NTAB, NTAB)])
    plsc.subcore_barrier()
    cps = [pltpu.async_copy(
               shared_parts.at[pl.ds(t * NTAB + sid * PSL, PSL)],
               pslice_v.at[t], sem)
           for t in range(NS)]
    for cp in cps:
        cp.wait()
    for jj in range(PSL // LANES):
        acc = pslice_v[0, pl.ds(jj * LANES, LANES)]
        for t in range(1, NS):
            acc = acc + pslice_v[t, pl.ds(jj * LANES, LANES)]
        tabslice_v[pl.ds(jj * LANES, LANES)] = acc
    pltpu.sync_copy(tabslice_v, shared_tab.at[pl.ds(sid * PSL, PSL)])
    plsc.subcore_barrier()
    pltpu.sync_copy(shared_tab, tab_v)

    npm = jnp.ones((LANES,), jnp.float32)
    for l in range(5):
        row = plsc.load_gather(tab_v, [l * CK + iota])
        s = jnp.sum(jnp.where(iota < K, row, 0.0))
        s = jnp.maximum(s, 1.0)
        npm = jnp.where(iota == l, s, npm)
    npm_v[...] = npm

    def sbody(j, carry):
        bi = j * LANES + iota
        d = plsc.load_gather(npm_v, [bi // CK])
        tab_v[pl.ds(j * LANES, LANES)] = tab_v[pl.ds(j * LANES, LANES)] / d
        return carry

    lax.fori_loop(0, NTAB // LANES, sbody, 0)

    boff = cid * EPW

    def wbody(i, carry):
        for u in range(2):
            o = boff + (2 * i + u) * LANES
            e = o + iota
            r = e // C
            c = e - r * C
            valf = vs_v[pl.ds(o, LANES)]
            vali = valf.astype(jnp.int32)
            lev = plsc.load_gather(lev_v, [r])
            b = lev * CK + c * K + vali
            t = plsc.load_gather(tab_v, [b])
            vs_v[pl.ds(o, LANES)] = valf * t
        return carry

    lax.fori_loop(0, NCHUNK // 2, wbody, 0)
    pltpu.sync_copy(vs_v.at[pl.ds(boff, EPW)],
                    out_hbm.at[pl.ds(wid * EPW, EPW)])


_fused = pl.kernel(
    _fused_body,
    out_type=jax.ShapeDtypeStruct((B * C,), jnp.float32),
    mesh=_mesh,
    compiler_params=_params,
    scratch_types=[
        pltpu.VMEM((EPT,), jnp.float32),
        pltpu.VMEM((RPT,), jnp.int32),
        pltpu.VMEM((NTAB,), jnp.float32),
        pltpu.VMEM((NS, PSL), jnp.float32),
        pltpu.VMEM((PSL,), jnp.float32),
        pltpu.VMEM((NTAB,), jnp.float32),
        pltpu.VMEM((LANES,), jnp.float32),
        pltpu.VMEM_SHARED((NS * NTAB,), jnp.float32),
        pltpu.VMEM_SHARED((NTAB,), jnp.float32),
        pltpu.SemaphoreType.DMA,
    ],
)


def _lstm_kernel(x_ref, wih_ref, bih_ref, bhh_ref, wfc_ref, bfc_ref, out_ref):
    x = x_ref[...]
    gates = lax.dot_general(x, wih_ref[...], (((1,), (1,)), ((), ())),
                            preferred_element_type=jnp.float32)
    gates = gates + bih_ref[...] + bhh_ref[...]
    i_g = jax.nn.sigmoid(gates[:, 0:H])
    g_g = jnp.tanh(gates[:, 2 * H:3 * H])
    o_g = jax.nn.sigmoid(gates[:, 3 * H:4 * H])
    h = o_g * jnp.tanh(i_g * g_g)
    out_ref[...] = lax.dot_general(h, wfc_ref[...], (((1,), (1,)), ((), ())),
                                   preferred_element_type=jnp.float32) + bfc_ref[...]


def kernel(VS, CC, Level, Depart, W_ih, W_hh, b_ih, b_hh, W_fc, b_fc):
    vs_flat = VS.reshape(B * C)
    lev_i = Level.astype(jnp.int32)
    w_flat = _fused(vs_flat, lev_i)
    x = w_flat.reshape(B, C)

    vs_feat = pl.pallas_call(
        _lstm_kernel,
        grid=(NB,),
        in_specs=[pl.BlockSpec((BLK, C), lambda i: (i, 0)),
                  pl.BlockSpec((G4, C), lambda i: (0, 0)),
                  pl.BlockSpec((1, G4), lambda i: (0, 0)),
                  pl.BlockSpec((1, G4), lambda i: (0, 0)),
                  pl.BlockSpec((OUT, H), lambda i: (0, 0)),
                  pl.BlockSpec((1, OUT), lambda i: (0, 0))],
        out_specs=pl.BlockSpec((BLK, OUT), lambda i: (i, 0)),
        out_shape=jax.ShapeDtypeStruct((B, OUT), jnp.float32),
    )(x, W_ih, b_ih.reshape(1, G4), b_hh.reshape(1, G4),
      W_fc, b_fc.reshape(1, OUT))

    return (jnp.squeeze(CC, axis=1), vs_feat, Level, Depart)
